# baseline (speedup 1.0000x reference)
import functools

import jax
import jax.numpy as jnp
from jax import lax
from jax.experimental import pallas as pl
from jax.experimental.pallas import tpu as pltpu
from jax.experimental.pallas import tpu_sc as plsc

N_NODES = 10000
N_EDGES = 320000
IN_FEATS = 128
H_FEATS = 128
NUM_CLASSES = 16

NP = 10240
NC = 2
NS = 16
NT = NC * NS
ROWS_PER_TILE = NP // NS
NSLOT = 4

B1, CHUNKS1 = 64, 316
B2, CHUNKS2 = 128, 80
EP1 = NS * CHUNKS1 * B1
EP2 = NT * CHUNKS2 * B2
WD = 16
HW = IN_FEATS // 2


def _make_sc_agg1():
  mesh = plsc.VectorSubcoreMesh(core_axis_name="c", subcore_axis_name="s")

  @functools.partial(
      pl.kernel,
      out_type=[
          jax.ShapeDtypeStruct((NC, NP, HW), jnp.float32),
          jax.ShapeDtypeStruct((NC, NP, WD), jnp.float32),
      ],
      mesh=mesh,
      compiler_params=pltpu.CompilerParams(use_tc_tiling_on_sc=False),
      scratch_types=[
          pltpu.VMEM((CHUNKS1, B1), jnp.int32),
          [pltpu.VMEM((B1,), jnp.int32) for _ in range(NSLOT)],
          [pltpu.VMEM((B1,), jnp.int32) for _ in range(NSLOT)],
          [pltpu.VMEM((B1, HW), jnp.float32) for _ in range(NSLOT)],
          pltpu.VMEM((B1, WD), jnp.float32),
          pltpu.VMEM_SHARED((NP, HW), jnp.float32),
          pltpu.VMEM_SHARED((NP, HW), jnp.float32),
          pltpu.VMEM_SHARED((NP, WD), jnp.float32),
          [pltpu.SemaphoreType.DMA for _ in range(NSLOT)],
          [pltpu.SemaphoreType.DMA for _ in range(NSLOT)],
          [pltpu.SemaphoreType.DMA for _ in range(NSLOT)],
      ],
  )
  def agg1_body(pk_hbm, x_hbm, out_hbm, deg_hbm, pidx, sidx, didx, rows,
                ones, table, acc, dacc, gsem, ssem, dsem):
    cid = lax.axis_index("c")
    sid = lax.axis_index("s")
    base = sid * ROWS_PER_TILE

    tbl_slab = table.at[pl.ds(base, ROWS_PER_TILE)]
    pltpu.async_copy(x_hbm.at[cid, pl.ds(base, ROWS_PER_TILE)], tbl_slab,
                     gsem[0])
    pltpu.async_copy(pk_hbm.at[sid], pidx, gsem[1])

    def unpack(j, k):
      for c in range(B1 // 16):
        v = pidx[j, pl.ds(c * 16, 16)]
        sidx[k][pl.ds(c * 16, 16)] = lax.shift_right_logical(v, 14)
        didx[k][pl.ds(c * 16, 16)] = lax.bitwise_and(v, 16383)

    zero = jnp.zeros((16,), jnp.float32)
    one = jnp.full((16,), 1.0, jnp.float32)
    for r in range(B1):
      for c in range(HW // 16):
        rows[0][r, pl.ds(c * 16, 16)] = zero
      ones[r, pl.ds(0, WD)] = zero

    nzb = ROWS_PER_TILE // B1
    for i in range(nzb):
      k = i % NSLOT
      if i >= NSLOT:
        pltpu.make_async_copy(
            rows[0], acc.at[pl.ds(base + (i - NSLOT) * B1, B1)],
            ssem[k]).wait()
        pltpu.make_async_copy(
            ones, dacc.at[pl.ds(base + (i - NSLOT) * B1, B1)],
            dsem[k]).wait()
      pltpu.async_copy(rows[0], acc.at[pl.ds(base + i * B1, B1)], ssem[k])
      pltpu.async_copy(ones, dacc.at[pl.ds(base + i * B1, B1)], dsem[k])
    for i in range(nzb - NSLOT, nzb):
      k = i % NSLOT
      pltpu.make_async_copy(rows[0], acc.at[pl.ds(base + i * B1, B1)],
                            ssem[k]).wait()
      pltpu.make_async_copy(ones, dacc.at[pl.ds(base + i * B1, B1)],
                            dsem[k]).wait()
    for r in range(B1):
      ones[r, pl.ds(0, WD)] = one
    pltpu.make_async_copy(x_hbm.at[cid, pl.ds(base, ROWS_PER_TILE)],
                          tbl_slab, gsem[0]).wait()
    pltpu.make_async_copy(pk_hbm.at[sid], pidx, gsem[1]).wait()
    plsc.subcore_barrier()

    def gather_start(k):
      pltpu.async_copy(table.at[sidx[k]], rows[k], gsem[k])

    def gather_wait(k):
      pltpu.make_async_copy(table.at[sidx[k]], rows[k], gsem[k]).wait()

    def scatter_start(k):
      pltpu.async_copy(rows[k], acc.at[didx[k]], ssem[k], add=True)

      @pl.when(cid == (k % 2))
      def _():
        pltpu.async_copy(ones, dacc.at[didx[k]], dsem[k], add=True)

    def scatter_wait(k):
      pltpu.make_async_copy(rows[k], acc.at[didx[k]], ssem[k]).wait()

      @pl.when(cid == (k % 2))
      def _():
        pltpu.make_async_copy(ones, dacc.at[didx[k]], dsem[k]).wait()

    nr = CHUNKS1 // NSLOT
    for k in range(NSLOT):
      unpack(k, k)
      gather_start(k)

    def edge_round(g, _):
      for k in range(NSLOT):
        gather_wait(k)
        scatter_start(k)
      for k in range(NSLOT):
        scatter_wait(k)
        unpack(NSLOT * g + k + NSLOT, k)
        gather_start(k)
      return 0

    lax.fori_loop(0, nr - 1, edge_round, 0)
    for k in range(NSLOT):
      gather_wait(k)
      scatter_start(k)
    for k in range(NSLOT):
      scatter_wait(k)
    plsc.subcore_barrier()

    pltpu.sync_copy(acc.at[pl.ds(base, ROWS_PER_TILE)],
                    out_hbm.at[cid, pl.ds(base, ROWS_PER_TILE)])
    pltpu.sync_copy(dacc.at[pl.ds(base, ROWS_PER_TILE)],
                    deg_hbm.at[cid, pl.ds(base, ROWS_PER_TILE)])

  return agg1_body


def _make_sc_agg2(width, bsz, chunks):
  mesh = plsc.VectorSubcoreMesh(core_axis_name="c", subcore_axis_name="s")

  @functools.partial(
      pl.kernel,
      out_type=jax.ShapeDtypeStruct((NC, NP, width), jnp.float32),
      mesh=mesh,
      compiler_params=pltpu.CompilerParams(use_tc_tiling_on_sc=False),
      scratch_types=[
          pltpu.VMEM((chunks, bsz), jnp.int32),
          pltpu.VMEM((chunks, bsz), jnp.int32),
          [pltpu.VMEM((bsz, width), jnp.float32) for _ in range(NSLOT)],
          pltpu.VMEM_SHARED((NP, width), jnp.float32),
          pltpu.VMEM_SHARED((NP, width), jnp.float32),
          [pltpu.SemaphoreType.DMA for _ in range(NSLOT)],
          [pltpu.SemaphoreType.DMA for _ in range(NSLOT)],
      ],
  )
  def agg_body(src_hbm, dst_hbm, z_hbm, out_hbm, sidx, didx, rows, table,
               acc, gsem, ssem):
    cid = lax.axis_index("c")
    sid = lax.axis_index("s")
    wid = cid * NS + sid
    base = sid * ROWS_PER_TILE

    tbl_slab = table.at[pl.ds(base, ROWS_PER_TILE)]
    pltpu.async_copy(src_hbm.at[wid], sidx, gsem[0])
    pltpu.async_copy(dst_hbm.at[wid], didx, gsem[1])
    pltpu.async_copy(z_hbm.at[pl.ds(base, ROWS_PER_TILE)], tbl_slab, gsem[2])

    zero = jnp.zeros((16,), jnp.float32)
    for r in range(bsz):
      for c in range(width // 16):
        rows[0][r, pl.ds(c * 16, 16)] = zero

    nzb = ROWS_PER_TILE // bsz
    for i in range(nzb):
      pltpu.async_copy(rows[0], acc.at[pl.ds(base + i * bsz, bsz)],
                       ssem[i % NSLOT])
    for i in range(nzb):
      pltpu.make_async_copy(rows[0], acc.at[pl.ds(base + i * bsz, bsz)],
                            ssem[i % NSLOT]).wait()
    pltpu.make_async_copy(src_hbm.at[wid], sidx, gsem[0]).wait()
    pltpu.make_async_copy(dst_hbm.at[wid], didx, gsem[1]).wait()
    pltpu.make_async_copy(z_hbm.at[pl.ds(base, ROWS_PER_TILE)], tbl_slab,
                          gsem[2]).wait()
    plsc.subcore_barrier()

    def gather_start(j, k):
      pltpu.async_copy(table.at[sidx.at[j]], rows[k], gsem[k])

    def gather_wait(j, k):
      pltpu.make_async_copy(table.at[sidx.at[j]], rows[k], gsem[k]).wait()

    def scatter_start(j, k):
      pltpu.async_copy(rows[k], acc.at[didx.at[j]], ssem[k], add=True)

    def scatter_wait(j, k):
      pltpu.make_async_copy(rows[k], acc.at[didx.at[j]], ssem[k]).wait()

    nr = chunks // NSLOT
    for k in range(NSLOT):
      gather_start(k, k)

    def edge_round(g, _):
      for k in range(NSLOT):
        j = NSLOT * g + k
        gather_wait(j, k)
        scatter_start(j, k)
      for k in range(NSLOT):
        j = NSLOT * g + k
        scatter_wait(j, k)
        gather_start(j + NSLOT, k)
      return 0

    lax.fori_loop(0, nr - 1, edge_round, 0)
    for k in range(NSLOT):
      j = NSLOT * (nr - 1) + k
      gather_wait(j, k)
      scatter_start(j, k)
    for k in range(NSLOT):
      j = NSLOT * (nr - 1) + k
      scatter_wait(j, k)
    plsc.subcore_barrier()

    pltpu.sync_copy(acc.at[pl.ds(base, ROWS_PER_TILE)],
                    out_hbm.at[cid, pl.ds(base, ROWS_PER_TILE)])

  return agg_body


_agg1 = _make_sc_agg1()
_agg2 = _make_sc_agg2(NUM_CLASSES, B2, CHUNKS2)

BLK = 2048


def _dense1_body(x0_ref, x1_ref, a0_ref, a1_ref, d0_ref, d1_ref, ws1_ref,
                 wn1_ref, b1_ref, wn2_ref, ws2_ref, b2_ref, z2_ref, p2_ref,
                 rdeg_ref):
  x = jnp.concatenate([x0_ref[0], x1_ref[0]], axis=1)
  a = jnp.concatenate([a0_ref[0], a1_ref[0]], axis=1)
  deg = jnp.maximum(d0_ref[0][:, 0:1] + d1_ref[0][:, 0:1], 1.0)
  rdeg = 1.0 / deg
  nbar = a * rdeg
  h1 = x @ ws1_ref[...] + nbar @ wn1_ref[...] + b1_ref[...]
  h1 = jnp.maximum(h1, 0.0)
  z2_ref[...] = h1 @ wn2_ref[...]
  p2_ref[...] = h1 @ ws2_ref[...] + b2_ref[...]
  rdeg_ref[...] = jnp.broadcast_to(rdeg, (BLK, NUM_CLASSES))


def _dense1(x_sc, agg1, deg, w_self1, w_neigh1, b1, w_neigh2, w_self2, b2):
  grid = (NP // BLK,)
  return pl.pallas_call(
      _dense1_body,
      grid=grid,
      in_specs=[
          pl.BlockSpec((1, BLK, HW), lambda i: (0, i, 0)),
          pl.BlockSpec((1, BLK, HW), lambda i: (1, i, 0)),
          pl.BlockSpec((1, BLK, HW), lambda i: (0, i, 0)),
          pl.BlockSpec((1, BLK, HW), lambda i: (1, i, 0)),
          pl.BlockSpec((1, BLK, WD), lambda i: (0, i, 0)),
          pl.BlockSpec((1, BLK, WD), lambda i: (1, i, 0)),
          pl.BlockSpec((IN_FEATS, H_FEATS), lambda i: (0, 0)),
          pl.BlockSpec((IN_FEATS, H_FEATS), lambda i: (0, 0)),
          pl.BlockSpec((1, H_FEATS), lambda i: (0, 0)),
          pl.BlockSpec((H_FEATS, NUM_CLASSES), lambda i: (0, 0)),
          pl.BlockSpec((H_FEATS, NUM_CLASSES), lambda i: (0, 0)),
          pl.BlockSpec((1, NUM_CLASSES), lambda i: (0, 0)),
      ],
      out_specs=[
          pl.BlockSpec((BLK, NUM_CLASSES), lambda i: (i, 0)),
          pl.BlockSpec((BLK, NUM_CLASSES), lambda i: (i, 0)),
          pl.BlockSpec((BLK, NUM_CLASSES), lambda i: (i, 0)),
      ],
      out_shape=[
          jax.ShapeDtypeStruct((NP, NUM_CLASSES), jnp.float32),
          jax.ShapeDtypeStruct((NP, NUM_CLASSES), jnp.float32),
          jax.ShapeDtypeStruct((NP, NUM_CLASSES), jnp.float32),
      ],
  )(x_sc, x_sc, agg1, agg1, deg, deg, w_self1, w_neigh1, b1, w_neigh2,
    w_self2, b2)


BLK2 = 2000


def _dense2_body(p2_ref, g0_ref, g1_ref, rdeg_ref, out_ref):
  out_ref[...] = p2_ref[...] + (g0_ref[0] + g1_ref[0]) * rdeg_ref[...]


def _dense2(p2, agg2, rdeg):
  grid = (N_NODES // BLK2,)
  return pl.pallas_call(
      _dense2_body,
      grid=grid,
      in_specs=[
          pl.BlockSpec((BLK2, NUM_CLASSES), lambda i: (i, 0)),
          pl.BlockSpec((1, BLK2, NUM_CLASSES), lambda i: (0, i, 0)),
          pl.BlockSpec((1, BLK2, NUM_CLASSES), lambda i: (1, i, 0)),
          pl.BlockSpec((BLK2, NUM_CLASSES), lambda i: (i, 0)),
      ],
      out_specs=pl.BlockSpec((BLK2, NUM_CLASSES), lambda i: (i, 0)),
      out_shape=jax.ShapeDtypeStruct((N_NODES, NUM_CLASSES), jnp.float32),
  )(p2, agg2, agg2, rdeg)


def _pad_edges(src, dst, ep, nsplit, nchunks, bsz):
  pad = ep - N_EDGES
  src_p = jnp.concatenate([src, jnp.zeros((pad,), jnp.int32)])
  dst_p = jnp.concatenate([dst, jnp.full((pad,), NP - 1, jnp.int32)])
  return src_p.reshape(nsplit, nchunks, bsz), dst_p.reshape(nsplit, nchunks, bsz)


@jax.jit
def _run(in_feat, edge_index, w_self1, w_neigh1, b1, w_self2, w_neigh2, b2):
  src = edge_index[0].astype(jnp.int32)
  dst = edge_index[1].astype(jnp.int32)
  pk = src * 16384 + dst
  pk1 = jnp.concatenate(
      [pk, jnp.full((EP1 - N_EDGES,), NP - 1, jnp.int32)]
  ).reshape(NS, CHUNKS1, B1)
  src2, dst2 = _pad_edges(src, dst, EP2, NT, CHUNKS2, B2)
  x_pad = jnp.pad(in_feat, ((0, NP - N_NODES), (0, 0)))
  x_sc = x_pad.reshape(NP, NC, HW).transpose(1, 0, 2)

  a1, dg = _agg1(pk1, x_sc)
  z2, p2, rdeg = _dense1(x_sc, a1, dg, w_self1, w_neigh1,
                         b1.reshape(1, H_FEATS), w_neigh2, w_self2,
                         b2.reshape(1, NUM_CLASSES))
  a2 = _agg2(src2, dst2, z2)
  return _dense2(p2, a2, rdeg)


def kernel(in_feat, edge_index, W_self1, W_neigh1, b1, W_self2, W_neigh2, b2):
  return _run(in_feat, edge_index, W_self1, W_neigh1, b1, W_self2, W_neigh2,
              b2)
